# initial kernel scaffold (unmeasured)
import jax
import jax.numpy as jnp
from jax import lax
from jax.experimental import pallas as pl
from jax.experimental.pallas import tpu as pltpu

C = 320
E_LOCAL = 4
D = 1024
F = 2048


def _moe_body(disp_mine_ref, disp_send_ref, w1_ref, w2_ref, out_ref,
              recv_ref, sendout_ref, sem_s1, sem_r1, sem_s2, sem_r2):
    my_x = lax.axis_index("x")
    my_y = lax.axis_index("y")
    partner = (1 - my_x, my_y)

    barrier_sem = pltpu.get_barrier_semaphore()
    pl.semaphore_signal(barrier_sem, inc=1, device_id=partner,
                        device_id_type=pl.DeviceIdType.MESH)
    pl.semaphore_wait(barrier_sem, 1)

    rdma1 = pltpu.make_async_remote_copy(
        src_ref=disp_send_ref,
        dst_ref=recv_ref,
        send_sem=sem_s1,
        recv_sem=sem_r1,
        device_id=partner,
        device_id_type=pl.DeviceIdType.MESH,
    )
    rdma1.start()

    for e in range(E_LOCAL):
        xa = disp_mine_ref[e]
        h = jnp.maximum(
            jnp.dot(xa, w1_ref[e], preferred_element_type=jnp.float32), 0.0
        ).astype(jnp.bfloat16)
        y = jnp.dot(h, w2_ref[e], preferred_element_type=jnp.float32)
        out_ref[0, e] = y.astype(jnp.bfloat16)

    rdma1.wait()

    for e in range(E_LOCAL):
        xa = recv_ref[e]
        h = jnp.maximum(
            jnp.dot(xa, w1_ref[e], preferred_element_type=jnp.float32), 0.0
        ).astype(jnp.bfloat16)
        y = jnp.dot(h, w2_ref[e], preferred_element_type=jnp.float32)
        sendout_ref[e] = y.astype(jnp.bfloat16)

    rdma2 = pltpu.make_async_remote_copy(
        src_ref=sendout_ref,
        dst_ref=out_ref.at[1],
        send_sem=sem_s2,
        recv_sem=sem_r2,
        device_id=partner,
        device_id_type=pl.DeviceIdType.MESH,
    )
    rdma2.start()
    rdma2.wait()


def kernel(x, assign, W1, W2):
    t, d = x.shape
    p = lax.axis_index("x")

    e_rel = jnp.remainder(assign + 8 - 4 * p, 8)
    oh = jax.nn.one_hot(e_rel, 8, dtype=jnp.int32)
    ranks = jnp.cumsum(oh, axis=0) - oh
    pos = jnp.take_along_axis(ranks, e_rel[:, None], axis=1)[:, 0]

    disp = jnp.zeros((8, C, d), jnp.bfloat16)
    disp = disp.at[e_rel, pos].set(x.astype(jnp.bfloat16), mode="drop")

    out = pl.pallas_call(
        _moe_body,
        out_shape=jax.ShapeDtypeStruct((2, E_LOCAL, C, d), jnp.bfloat16),
        in_specs=[
            pl.BlockSpec(memory_space=pltpu.VMEM),
            pl.BlockSpec(memory_space=pltpu.VMEM),
            pl.BlockSpec(memory_space=pltpu.VMEM),
            pl.BlockSpec(memory_space=pltpu.VMEM),
        ],
        out_specs=pl.BlockSpec(memory_space=pltpu.VMEM),
        scratch_shapes=[
            pltpu.VMEM((E_LOCAL, C, d), jnp.bfloat16),
            pltpu.VMEM((E_LOCAL, C, d), jnp.bfloat16),
            pltpu.SemaphoreType.DMA,
            pltpu.SemaphoreType.DMA,
            pltpu.SemaphoreType.DMA,
            pltpu.SemaphoreType.DMA,
        ],
        compiler_params=pltpu.CompilerParams(collective_id=0),
    )(
        disp[:E_LOCAL],
        disp[E_LOCAL:],
        W1.astype(jnp.bfloat16),
        W2.astype(jnp.bfloat16),
    )

    combined = out.reshape(2 * E_LOCAL, C, d)
    return combined[e_rel, pos].astype(jnp.float32)


# baseline (device time: 186916 ns/iter reference)
import jax
import jax.numpy as jnp
from jax import lax
from jax.experimental import pallas as pl
from jax.experimental.pallas import tpu as pltpu

C = 320
E_LOCAL = 4
D = 1024
F = 2048


def _moe_body(disp_mine_ref, disp_send_ref, w1_ref, w2_ref, out_ref,
              recv_ref, sendout_ref, sem_s1, sem_r1, sem_s2, sem_r2):
    my_x = lax.axis_index("x")
    my_y = lax.axis_index("y")
    partner = (1 - my_x, my_y)

    barrier_sem = pltpu.get_barrier_semaphore()
    pl.semaphore_signal(barrier_sem, inc=1, device_id=partner,
                        device_id_type=pl.DeviceIdType.MESH)
    pl.semaphore_wait(barrier_sem, 1)

    rdma1 = pltpu.make_async_remote_copy(
        src_ref=disp_send_ref,
        dst_ref=recv_ref,
        send_sem=sem_s1,
        recv_sem=sem_r1,
        device_id=partner,
        device_id_type=pl.DeviceIdType.MESH,
    )
    rdma1.start()

    for e in range(E_LOCAL):
        xa = disp_mine_ref[e]
        h = jnp.maximum(
            jnp.dot(xa, w1_ref[e], preferred_element_type=jnp.float32), 0.0
        ).astype(jnp.bfloat16)
        y = jnp.dot(h, w2_ref[e], preferred_element_type=jnp.float32)
        out_ref[0, e] = y.astype(jnp.bfloat16)

    rdma1.wait()

    for e in range(E_LOCAL):
        xa = recv_ref[e]
        h = jnp.maximum(
            jnp.dot(xa, w1_ref[e], preferred_element_type=jnp.float32), 0.0
        ).astype(jnp.bfloat16)
        y = jnp.dot(h, w2_ref[e], preferred_element_type=jnp.float32)
        sendout_ref[e] = y.astype(jnp.bfloat16)

    rdma2 = pltpu.make_async_remote_copy(
        src_ref=sendout_ref,
        dst_ref=out_ref.at[1],
        send_sem=sem_s2,
        recv_sem=sem_r2,
        device_id=partner,
        device_id_type=pl.DeviceIdType.MESH,
    )
    rdma2.start()
    rdma2.wait()


def kernel(x, assign, W1, W2):
    t, d = x.shape
    p = lax.axis_index("x")

    e_rel = jnp.remainder(assign + 8 - 4 * p, 8)
    oh = jax.nn.one_hot(e_rel, 8, dtype=jnp.int32)
    ranks = jnp.cumsum(oh, axis=0) - oh
    pos = jnp.take_along_axis(ranks, e_rel[:, None], axis=1)[:, 0]

    disp = jnp.zeros((8, C, d), jnp.bfloat16)
    disp = disp.at[e_rel, pos].set(x.astype(jnp.bfloat16), mode="drop")

    out = pl.pallas_call(
        _moe_body,
        out_shape=jax.ShapeDtypeStruct((2, E_LOCAL, C, d), jnp.bfloat16),
        in_specs=[
            pl.BlockSpec(memory_space=pltpu.VMEM),
            pl.BlockSpec(memory_space=pltpu.VMEM),
            pl.BlockSpec(memory_space=pltpu.VMEM),
            pl.BlockSpec(memory_space=pltpu.VMEM),
        ],
        out_specs=pl.BlockSpec(memory_space=pltpu.VMEM),
        scratch_shapes=[
            pltpu.VMEM((E_LOCAL, C, d), jnp.bfloat16),
            pltpu.VMEM((E_LOCAL, C, d), jnp.bfloat16),
            pltpu.SemaphoreType.DMA,
            pltpu.SemaphoreType.DMA,
            pltpu.SemaphoreType.DMA,
            pltpu.SemaphoreType.DMA,
        ],
        compiler_params=pltpu.CompilerParams(
            collective_id=0,
            vmem_limit_bytes=63 * 1024 * 1024,
        ),
    )(
        disp[:E_LOCAL],
        disp[E_LOCAL:],
        W1.astype(jnp.bfloat16),
        W2.astype(jnp.bfloat16),
    )

    combined = out.reshape(2 * E_LOCAL, C, d)
    return combined[e_rel, pos].astype(jnp.float32)


# device time: 166113 ns/iter; 1.1252x vs baseline; 1.1252x over previous
import jax
import jax.numpy as jnp
from jax import lax
from jax.experimental import pallas as pl
from jax.experimental.pallas import tpu as pltpu

C = 320
E_LOCAL = 4
D = 1024
F = 2048
T = 2048
CHUNK = 256


def _moe_body(disp_mine_ref, disp_send_ref, w1_ref, w2_ref, key_ref,
              out_ref, recv_ref, flat_ref, sendout_ref,
              sem_s1, sem_r1, sem_s2, sem_r2):
    my_x = lax.axis_index("x")
    my_y = lax.axis_index("y")
    partner = (1 - my_x, my_y)

    barrier_sem = pltpu.get_barrier_semaphore()
    pl.semaphore_signal(barrier_sem, inc=1, device_id=partner,
                        device_id_type=pl.DeviceIdType.MESH)
    pl.semaphore_wait(barrier_sem, 1)

    rdma1 = pltpu.make_async_remote_copy(
        src_ref=disp_send_ref,
        dst_ref=recv_ref,
        send_sem=sem_s1,
        recv_sem=sem_r1,
        device_id=partner,
        device_id_type=pl.DeviceIdType.MESH,
    )
    rdma1.start()

    def expert_ffn(xa, e):
        h = jnp.maximum(
            jnp.dot(xa, w1_ref[e], preferred_element_type=jnp.float32), 0.0
        ).astype(jnp.bfloat16)
        y = jnp.dot(h, w2_ref[e], preferred_element_type=jnp.float32)
        return y.astype(jnp.bfloat16)

    for e in range(E_LOCAL):
        flat_ref[pl.ds(e * C, C), :] = expert_ffn(disp_mine_ref[e], e)

    rdma1.wait()

    for e in range(E_LOCAL):
        sendout_ref[pl.ds(e * C, C), :] = expert_ffn(recv_ref[e], e)

    rdma2 = pltpu.make_async_remote_copy(
        src_ref=sendout_ref,
        dst_ref=flat_ref.at[pl.ds(E_LOCAL * C, E_LOCAL * C)],
        send_sem=sem_s2,
        recv_sem=sem_r2,
        device_id=partner,
        device_id_type=pl.DeviceIdType.MESH,
    )
    rdma2.start()
    rdma2.wait()

    n_rows = 2 * E_LOCAL * C
    for i in range(T // CHUNK):
        k = key_ref[pl.ds(i * CHUNK, CHUNK)]
        col = lax.broadcasted_iota(jnp.int32, (CHUNK, n_rows), 1)
        g = (col == k[:, None]).astype(jnp.bfloat16)
        out_ref[pl.ds(i * CHUNK, CHUNK), :] = jnp.dot(
            g, flat_ref[:, :], preferred_element_type=jnp.float32
        )


def kernel(x, assign, W1, W2):
    t, d = x.shape
    p = lax.axis_index("x")

    e_rel = jnp.remainder(assign + 8 - 4 * p, 8)
    oh = jax.nn.one_hot(e_rel, 8, dtype=jnp.int32)
    ranks = jnp.cumsum(oh, axis=0) - oh
    pos = jnp.take_along_axis(ranks, e_rel[:, None], axis=1)[:, 0]
    key = (e_rel * C + pos).astype(jnp.int32)

    disp = jnp.zeros((8, C, d), jnp.bfloat16)
    disp = disp.at[e_rel, pos].set(x.astype(jnp.bfloat16), mode="drop")

    return pl.pallas_call(
        _moe_body,
        out_shape=jax.ShapeDtypeStruct((T, d), jnp.float32),
        in_specs=[
            pl.BlockSpec(memory_space=pltpu.VMEM),
            pl.BlockSpec(memory_space=pltpu.VMEM),
            pl.BlockSpec(memory_space=pltpu.VMEM),
            pl.BlockSpec(memory_space=pltpu.VMEM),
            pl.BlockSpec(memory_space=pltpu.VMEM),
        ],
        out_specs=pl.BlockSpec(memory_space=pltpu.VMEM),
        scratch_shapes=[
            pltpu.VMEM((E_LOCAL, C, d), jnp.bfloat16),
            pltpu.VMEM((2 * E_LOCAL * C, d), jnp.bfloat16),
            pltpu.VMEM((E_LOCAL * C, d), jnp.bfloat16),
            pltpu.SemaphoreType.DMA,
            pltpu.SemaphoreType.DMA,
            pltpu.SemaphoreType.DMA,
            pltpu.SemaphoreType.DMA,
        ],
        compiler_params=pltpu.CompilerParams(
            collective_id=0,
            vmem_limit_bytes=63 * 1024 * 1024,
        ),
    )(
        disp[:E_LOCAL],
        disp[E_LOCAL:],
        W1.astype(jnp.bfloat16),
        W2.astype(jnp.bfloat16),
        key,
    )


# device time: 78405 ns/iter; 2.3840x vs baseline; 2.1187x over previous
import jax
import jax.numpy as jnp
from jax import lax
from jax.experimental import pallas as pl
from jax.experimental.pallas import tpu as pltpu

C = 288
E_LOCAL = 4
D = 1024
F = 2048
T = 2048
CHUNK = 256


def _moe_body(xb_ref, key_ref, w1_any, w2_any, out_ref,
              send_disp, recv_ref, flat_ref, sendout_ref, w1s, w2s,
              w1sem, w2sem, sem_s1, sem_r1, sem_s2, sem_r2):
    my_x = lax.axis_index("x")
    my_y = lax.axis_index("y")
    partner = (1 - my_x, my_y)

    def w_copy(e):
        s = e % 2
        c1 = pltpu.make_async_copy(w1_any.at[e], w1s.at[s], w1sem.at[s])
        c2 = pltpu.make_async_copy(w2_any.at[e], w2s.at[s], w2sem.at[s])
        c1.start()
        c2.start()
        return c1, c2

    pend = w_copy(0)

    barrier_sem = pltpu.get_barrier_semaphore()
    pl.semaphore_signal(barrier_sem, inc=1, device_id=partner,
                        device_id_type=pl.DeviceIdType.MESH)
    pl.semaphore_wait(barrier_sem, 1)

    def gather_rows(base):
        k = key_ref[:] - base
        row = lax.broadcasted_iota(jnp.int32, (C, T), 0)
        return (row == k[None, :]).astype(jnp.float32)

    rdma1 = []
    for e in range(E_LOCAL):
        g = gather_rows((E_LOCAL + e) * C)
        send_disp[e] = jnp.dot(
            g, xb_ref[:, :], preferred_element_type=jnp.float32
        ).astype(jnp.bfloat16)
        r = pltpu.make_async_remote_copy(
            src_ref=send_disp.at[e],
            dst_ref=recv_ref.at[e],
            send_sem=sem_s1.at[e],
            recv_sem=sem_r1.at[e],
            device_id=partner,
            device_id_type=pl.DeviceIdType.MESH,
        )
        r.start()
        rdma1.append(r)

    def ffn(xa, s):
        h = jnp.maximum(
            jnp.dot(xa, w1s[s], preferred_element_type=jnp.float32), 0.0
        )
        y = jnp.dot(h, w2s[s], preferred_element_type=jnp.float32)
        return y.astype(jnp.bfloat16)

    rdma2 = []
    for e in range(E_LOCAL):
        c1, c2 = pend
        if e + 1 < E_LOCAL:
            pend = w_copy(e + 1)
        c1.wait()
        c2.wait()
        s = e % 2
        xa = jnp.dot(
            gather_rows(e * C), xb_ref[:, :],
            preferred_element_type=jnp.float32,
        )
        flat_ref[pl.ds(e * C, C), :] = ffn(xa, s)
        rdma1[e].wait()
        sendout_ref[pl.ds(e * C, C), :] = ffn(
            recv_ref[e].astype(jnp.float32), s
        )
        r = pltpu.make_async_remote_copy(
            src_ref=sendout_ref.at[pl.ds(e * C, C)],
            dst_ref=flat_ref.at[pl.ds((E_LOCAL + e) * C, C)],
            send_sem=sem_s2.at[e],
            recv_sem=sem_r2.at[e],
            device_id=partner,
            device_id_type=pl.DeviceIdType.MESH,
        )
        r.start()
        rdma2.append(r)

    half = E_LOCAL * C

    def combine(base, ncols, accumulate):
        for i in range(T // CHUNK):
            k = key_ref[pl.ds(i * CHUNK, CHUNK)] - base
            col = lax.broadcasted_iota(jnp.int32, (CHUNK, ncols), 1)
            g = (col == k[:, None]).astype(jnp.bfloat16)
            part = jnp.dot(
                g, flat_ref[pl.ds(base, ncols), :],
                preferred_element_type=jnp.float32,
            )
            if accumulate:
                out_ref[pl.ds(i * CHUNK, CHUNK), :] += part
            else:
                out_ref[pl.ds(i * CHUNK, CHUNK), :] = part

    combine(0, half, accumulate=False)
    rdma2[0].wait()
    rdma2[1].wait()
    combine(half, 2 * C, accumulate=True)
    rdma2[2].wait()
    rdma2[3].wait()
    combine(half + 2 * C, 2 * C, accumulate=True)


def kernel(x, assign, W1, W2):
    t, d = x.shape
    p = lax.axis_index("x")

    e_rel = jnp.remainder(assign + 8 - 4 * p, 8)
    oh = jax.nn.one_hot(e_rel, 8, dtype=jnp.int32)
    ranks = jnp.cumsum(oh, axis=0) - oh
    pos = jnp.sum(ranks * oh, axis=1)
    key = (e_rel * C + pos).astype(jnp.int32)

    return pl.pallas_call(
        _moe_body,
        out_shape=jax.ShapeDtypeStruct((T, d), jnp.float32),
        in_specs=[
            pl.BlockSpec(memory_space=pltpu.VMEM),
            pl.BlockSpec(memory_space=pltpu.VMEM),
            pl.BlockSpec(memory_space=pl.ANY),
            pl.BlockSpec(memory_space=pl.ANY),
        ],
        out_specs=pl.BlockSpec(memory_space=pltpu.VMEM),
        scratch_shapes=[
            pltpu.VMEM((E_LOCAL, C, D), jnp.bfloat16),
            pltpu.VMEM((E_LOCAL, C, D), jnp.bfloat16),
            pltpu.VMEM((2 * E_LOCAL * C, D), jnp.bfloat16),
            pltpu.VMEM((E_LOCAL * C, D), jnp.bfloat16),
            pltpu.VMEM((2, D, F), jnp.float32),
            pltpu.VMEM((2, F, D), jnp.float32),
            pltpu.SemaphoreType.DMA((2,)),
            pltpu.SemaphoreType.DMA((2,)),
            pltpu.SemaphoreType.DMA((E_LOCAL,)),
            pltpu.SemaphoreType.DMA((E_LOCAL,)),
            pltpu.SemaphoreType.DMA((E_LOCAL,)),
            pltpu.SemaphoreType.DMA((E_LOCAL,)),
        ],
        compiler_params=pltpu.CompilerParams(
            collective_id=0,
            vmem_limit_bytes=63 * 1024 * 1024,
        ),
    )(x, key, W1, W2)
